# baseline (speedup 1.0000x reference)
import jax
import jax.numpy as jnp
from jax.experimental import pallas as pl
from jax.experimental.pallas import tpu as pltpu

BM = 512


def _linear_kernel(x_ref, wt_ref, b_ref, o_ref):
    o_ref[...] = jnp.broadcast_to(b_ref[...], o_ref.shape) + jnp.dot(
        x_ref[...], wt_ref[...], preferred_element_type=jnp.float32
    )


def kernel(encoding, W, b):
    B, T, D = encoding.shape
    L = W.shape[0]
    M = B * T
    x = encoding.reshape(M, D)
    wt = W.T
    b2 = b.reshape(1, L)

    out = pl.pallas_call(
        _linear_kernel,
        grid=(M // BM,),
        in_specs=[
            pl.BlockSpec((BM, D), lambda i: (i, 0)),
            pl.BlockSpec((D, L), lambda i: (0, 0)),
            pl.BlockSpec((1, L), lambda i: (0, 0)),
        ],
        out_specs=pl.BlockSpec((BM, L), lambda i: (i, 0)),
        out_shape=jax.ShapeDtypeStruct((M, L), jnp.float32),
        compiler_params=pltpu.CompilerParams(
            dimension_semantics=("parallel",),
        ),
    )(x, wt, b2)
    return out.reshape(B, T, L)


# speedup vs baseline: 1.1435x; 1.1435x over previous
import jax
import jax.numpy as jnp
from jax.experimental import pallas as pl
from jax.experimental.pallas import tpu as pltpu

BM = 2048
NS = 2


def _linear_kernel(*refs):
    x_refs = refs[:NS]
    wt_ref = refs[NS]
    b_ref = refs[NS + 1]
    o_ref = refs[NS + 2]
    dk = wt_ref.shape[0] // NS
    acc = jnp.broadcast_to(b_ref[...], o_ref.shape)
    for j in range(NS):
        acc = acc + jnp.dot(
            x_refs[j][...],
            wt_ref[j * dk:(j + 1) * dk, :],
            preferred_element_type=jnp.float32,
        )
    o_ref[...] = acc


def kernel(encoding, W, b):
    B, T, D = encoding.shape
    L = W.shape[0]
    M = B * T
    x = encoding.reshape(M, D)
    wt = W.T
    b2 = b.reshape(1, L)
    dk = D // NS

    in_specs = [
        pl.BlockSpec((BM, dk), lambda i, j=j: (i, j)) for j in range(NS)
    ] + [
        pl.BlockSpec((D, L), lambda i: (0, 0)),
        pl.BlockSpec((1, L), lambda i: (0, 0)),
    ]

    out = pl.pallas_call(
        _linear_kernel,
        grid=(M // BM,),
        in_specs=in_specs,
        out_specs=pl.BlockSpec((BM, L), lambda i: (i, 0)),
        out_shape=jax.ShapeDtypeStruct((M, L), jnp.float32),
        compiler_params=pltpu.CompilerParams(
            dimension_semantics=("parallel",),
        ),
    )(*([x] * NS), wt, b2)
    return out.reshape(B, T, L)


# speedup vs baseline: 1.1700x; 1.0232x over previous
import jax
import jax.numpy as jnp
from jax import lax
from jax.experimental import pallas as pl
from jax.experimental.pallas import tpu as pltpu

BM = 2048


def _linear_kernel(x_ref, w_ref, b_ref, o_ref):
    acc = lax.dot_general(
        x_ref[...], w_ref[...],
        dimension_numbers=(((1,), (1,)), ((), ())),
        preferred_element_type=jnp.float32,
    )
    o_ref[...] = acc + jnp.broadcast_to(b_ref[...], o_ref.shape)


def kernel(encoding, W, b):
    B, T, D = encoding.shape
    L = W.shape[0]
    M = B * T
    x = encoding.reshape(M, D)
    b2 = b.reshape(1, L)

    out = pl.pallas_call(
        _linear_kernel,
        grid=(M // BM,),
        in_specs=[
            pl.BlockSpec((BM, D), lambda i: (i, 0)),
            pl.BlockSpec((L, D), lambda i: (0, 0)),
            pl.BlockSpec((1, L), lambda i: (0, 0)),
        ],
        out_specs=pl.BlockSpec((BM, L), lambda i: (i, 0)),
        out_shape=jax.ShapeDtypeStruct((M, L), jnp.float32),
        compiler_params=pltpu.CompilerParams(
            dimension_semantics=("parallel",),
        ),
    )(x, W, b2)
    return out.reshape(B, T, L)


# speedup vs baseline: 1.5195x; 1.2986x over previous
import jax
import jax.numpy as jnp
from jax import lax
from jax.experimental import pallas as pl
from jax.experimental.pallas import tpu as pltpu

BT = 512


def _linear_kernel(x_ref, w_ref, b_ref, o_ref):
    nb, bt, d = x_ref.shape
    l = w_ref.shape[0]
    xm = x_ref[...].reshape(nb * bt, d)
    r = lax.dot_general(
        w_ref[...], xm,
        dimension_numbers=(((1,), (1,)), ((), ())),
        preferred_element_type=jnp.float32,
    )
    r = r + b_ref[...]
    o_ref[...] = r.reshape(l, nb, bt)


def kernel(encoding, W, b):
    B, T, D = encoding.shape
    L = W.shape[0]
    b2 = b.reshape(L, 1)

    out = pl.pallas_call(
        _linear_kernel,
        grid=(T // BT,),
        in_specs=[
            pl.BlockSpec((B, BT, D), lambda i: (0, i, 0)),
            pl.BlockSpec((L, D), lambda i: (0, 0)),
            pl.BlockSpec((L, 1), lambda i: (0, 0)),
        ],
        out_specs=pl.BlockSpec((L, B, BT), lambda i: (0, 0, i)),
        out_shape=jax.ShapeDtypeStruct((L, B, T), jnp.float32),
        compiler_params=pltpu.CompilerParams(
            dimension_semantics=("parallel",),
        ),
    )(encoding, W, b2)
    return out.transpose(1, 2, 0)


# speedup vs baseline: 1.5392x; 1.0130x over previous
import jax
import jax.numpy as jnp
from jax import lax
from jax.experimental import pallas as pl
from jax.experimental.pallas import tpu as pltpu

BT = 512


def _linear_kernel(x_ref, w_ref, b_ref, o_ref):
    nb, bt, d = x_ref.shape
    l = w_ref.shape[0]
    xm = x_ref[...].reshape(nb * bt, d)
    r = lax.dot_general(
        w_ref[...], xm,
        dimension_numbers=(((1,), (1,)), ((), ())),
        preferred_element_type=jnp.float32,
    )
    r = r + b_ref[...].T
    o_ref[...] = r.reshape(l, nb, bt)


def kernel(encoding, W, b):
    B, T, D = encoding.shape
    L = W.shape[0]
    b2 = b.reshape(1, L)

    out = pl.pallas_call(
        _linear_kernel,
        grid=(T // BT,),
        in_specs=[
            pl.BlockSpec((B, BT, D), lambda i: (0, i, 0)),
            pl.BlockSpec((L, D), lambda i: (0, 0)),
            pl.BlockSpec((1, L), lambda i: (0, 0)),
        ],
        out_specs=pl.BlockSpec((L, B, BT), lambda i: (0, 0, i)),
        out_shape=jax.ShapeDtypeStruct((L, B, T), jnp.float32),
        compiler_params=pltpu.CompilerParams(
            dimension_semantics=("parallel",),
        ),
    )(encoding, W, b2)
    return out.transpose(1, 2, 0)
